# Initial kernel scaffold; baseline (speedup 1.0000x reference)
import jax
import jax.numpy as jnp
from jax.experimental import pallas as pl


def kernel(x, edge_index, edge_type, W_rel1, W_root1, b1, W_rel2, W_root2, b2):
    raise NotImplementedError("write your pallas kernel here")


# speedup vs baseline: 16.0592x; 16.0592x over previous
import functools

import jax
import jax.numpy as jnp
from jax import lax
from jax.experimental import pallas as pl
from jax.experimental.pallas import tpu as pltpu
from jax.experimental.pallas import tpu_sc as plsc

N_NODES = 10000
N_REL = 8
N_EDGES = 320000

NC = 2
NS = 16
NW = NC * NS
LANES = 16

ET = N_EDGES // NW
KPAD = 81920
NPAD = 10240
ROWS_PER_TILE = NPAD // NS

_MESH = dict(core_axis_name="c", subcore_axis_name="s")


def _wid():
    return lax.axis_index("s") * NC + lax.axis_index("c")


def _fill(ref, n, value, dtype):
    vec = jnp.full((LANES,), value, dtype)

    def body(i, _):
        ref[pl.ds(i * LANES, LANES)] = vec
        return 0

    lax.fori_loop(0, n // LANES, body, 0)


def _count_body(etype, src, dst, part_out, skey_out, gkey_out,
                tbuf, sbuf, dbuf, kbuf, gbuf, ones, zbuf, cnt_sh):
    wid = _wid()
    sid = lax.axis_index("s")
    cid = lax.axis_index("c")

    CB = 128
    NCHUNK = N_EDGES // CB
    NITER = -(-NCHUNK // NW)

    _fill(ones, CB, 1.0, jnp.float32)
    _fill(zbuf, KPAD // NS, 0.0, jnp.float32)
    pltpu.sync_copy(zbuf, cnt_sh.at[pl.ds(sid * (KPAD // NS), KPAD // NS)])
    plsc.subcore_barrier()

    def chunk(i, _):
        cidx = i * NW + wid

        @pl.when(cidx < NCHUNK)
        def _():
            base = cidx * CB
            pltpu.sync_copy(etype.at[pl.ds(base, CB)], tbuf)
            pltpu.sync_copy(src.at[pl.ds(base, CB)], sbuf)
            pltpu.sync_copy(dst.at[pl.ds(base, CB)], dbuf)

            def body(j, _):
                sl = pl.ds(j * LANES, LANES)
                t = tbuf[sl] * N_NODES
                kbuf[sl] = t + dbuf[sl]
                gbuf[sl] = t + sbuf[sl]
                return 0

            lax.fori_loop(0, CB // LANES, body, 0)
            pltpu.sync_copy(kbuf, skey_out.at[pl.ds(base, CB)])
            pltpu.sync_copy(gbuf, gkey_out.at[pl.ds(base, CB)])
            pltpu.sync_copy(ones, cnt_sh.at[kbuf], add=True)

        return 0

    lax.fori_loop(0, NITER, chunk, 0)

    plsc.subcore_barrier()
    off = sid * (KPAD // NS)
    pltpu.sync_copy(cnt_sh.at[pl.ds(off, KPAD // NS)],
                    part_out.at[cid, pl.ds(off, KPAD // NS)])


def _sc_count(etype, src, dst):
    mesh = plsc.VectorSubcoreMesh(**_MESH)
    f = pl.kernel(
        _count_body,
        out_type=(
            jax.ShapeDtypeStruct((NC, KPAD), jnp.float32),
            jax.ShapeDtypeStruct((N_EDGES,), jnp.int32),
            jax.ShapeDtypeStruct((N_EDGES,), jnp.int32),
        ),
        mesh=mesh,
        compiler_params=pltpu.CompilerParams(needs_layout_passes=False),
        scratch_types=[
            pltpu.VMEM((128,), jnp.int32),
            pltpu.VMEM((128,), jnp.int32),
            pltpu.VMEM((128,), jnp.int32),
            pltpu.VMEM((128,), jnp.int32),
            pltpu.VMEM((128,), jnp.int32),
            pltpu.VMEM((128,), jnp.float32),
            pltpu.VMEM((KPAD // NS,), jnp.float32),
            pltpu.VMEM_SHARED((KPAD,), jnp.float32),
        ],
    )
    return f(etype, src, dst)


def _scale_body(part, skey, c_out, abuf, bbuf, kbuf, cbuf, tab, cnt_sh):
    sid = lax.axis_index("s")
    wid = _wid()

    W = KPAD // NS
    off = sid * W
    pltpu.sync_copy(part.at[0, pl.ds(off, W)], abuf)
    pltpu.sync_copy(part.at[1, pl.ds(off, W)], bbuf)

    def body(i, _):
        sl = pl.ds(i * LANES, LANES)
        abuf[sl] = abuf[sl] + bbuf[sl]
        return 0

    lax.fori_loop(0, W // LANES, body, 0)
    pltpu.sync_copy(abuf, cnt_sh.at[pl.ds(off, W)])
    plsc.subcore_barrier()

    pltpu.sync_copy(cnt_sh, tab)

    CB = 2000
    one = jnp.full((LANES,), 1.0, jnp.float32)

    for k in range(ET // CB):
        base = wid * ET + k * CB
        pltpu.sync_copy(skey.at[pl.ds(base, CB)], kbuf)

        def body(i, _):
            sl = pl.ds(i * LANES, LANES)
            cnt = plsc.load_gather(tab, [kbuf[sl]])
            cbuf[sl] = one / jnp.maximum(cnt, one)
            return 0

        lax.fori_loop(0, CB // LANES, body, 0)
        pltpu.sync_copy(cbuf, c_out.at[pl.ds(base, CB)])


def _sc_scale(part, skey):
    mesh = plsc.VectorSubcoreMesh(**_MESH)
    W = KPAD // NS
    f = pl.kernel(
        _scale_body,
        out_type=jax.ShapeDtypeStruct((N_EDGES,), jnp.float32),
        mesh=mesh,
        compiler_params=pltpu.CompilerParams(needs_layout_passes=False),
        scratch_types=[
            pltpu.VMEM((W,), jnp.float32),
            pltpu.VMEM((W,), jnp.float32),
            pltpu.VMEM((2000,), jnp.int32),
            pltpu.VMEM((2000,), jnp.float32),
            pltpu.VMEM((KPAD,), jnp.float32),
            pltpu.VMEM_SHARED((KPAD,), jnp.float32),
        ],
    )
    return f(part, skey)


def _agg_body(D, xr, gkey, dst, c, part_out,
              gbuf, dbuf, cbuf, rows, sem, acc_sh):
    wid = _wid()
    sid = lax.axis_index("s")
    cid = lax.axis_index("c")
    DV = D // LANES
    B = 128
    NCHUNK = N_EDGES // B
    NITER = -(-NCHUNK // NW)

    def zbody(e, _):
        for j in range(DV):
            rows[e, pl.ds(j * LANES, LANES)] = jnp.zeros((LANES,), jnp.float32)
        return 0

    lax.fori_loop(0, B, zbody, 0)
    for z in range(ROWS_PER_TILE // B):
        pltpu.sync_copy(rows,
                        acc_sh.at[pl.ds(sid * ROWS_PER_TILE + z * B, B), :])
    plsc.subcore_barrier()

    def chunk(i, _):
        cidx = i * NW + wid

        @pl.when(cidx < NCHUNK)
        def _():
            base = cidx * B
            pltpu.sync_copy(gkey.at[pl.ds(base, B)], gbuf)
            cp = pltpu.async_copy(xr.at[gbuf], rows, sem)
            pltpu.sync_copy(c.at[pl.ds(base, B)], cbuf)
            pltpu.sync_copy(dst.at[pl.ds(base, B)], dbuf)
            cp.wait()

            def body(e, _):
                cv = plsc.load_gather(cbuf, [jnp.full((LANES,), e, jnp.int32)])
                for j in range(DV):
                    sl = pl.ds(j * LANES, LANES)
                    rows[e, sl] = rows[e, sl] * cv
                return 0

            lax.fori_loop(0, B, body, 0)
            pltpu.sync_copy(rows, acc_sh.at[dbuf], add=True)

        return 0

    lax.fori_loop(0, NITER, chunk, 0)

    plsc.subcore_barrier()
    roff = sid * ROWS_PER_TILE
    for z in range(ROWS_PER_TILE // B):
        pltpu.sync_copy(acc_sh.at[pl.ds(roff + z * B, B), :],
                        part_out.at[cid, pl.ds(roff + z * B, B), :])


def _sc_agg(xr, gkey, dst, c, D):
    B = 128
    mesh = plsc.VectorSubcoreMesh(**_MESH)
    f = pl.kernel(
        functools.partial(_agg_body, D),
        out_type=jax.ShapeDtypeStruct((NC, NPAD, D), jnp.float32),
        mesh=mesh,
        compiler_params=pltpu.CompilerParams(needs_layout_passes=False),
        scratch_types=[
            pltpu.VMEM((B,), jnp.int32),
            pltpu.VMEM((B,), jnp.int32),
            pltpu.VMEM((B,), jnp.float32),
            pltpu.VMEM((B, D), jnp.float32),
            pltpu.SemaphoreType.DMA,
            pltpu.VMEM_SHARED((NPAD, D), jnp.float32),
        ],
    )
    return f(xr, gkey, dst, c)


def _slot_body(xr, gkey, dst, c, part_out,
               kbuf, gbuf, sbuf, dbuf, cbuf, rows, srows, sem, acc_sh):
    wid = _wid()
    sid = lax.axis_index("s")
    cid = lax.axis_index("c")
    B = 128
    NCHUNK = N_EDGES // B
    NITER = -(-NCHUNK // NW)
    iota = lax.iota(jnp.int32, LANES)
    zvec = jnp.zeros((LANES,), jnp.float32)

    def zbody(e, _):
        for j in range(8):
            sl = pl.ds(j * LANES, LANES)
            rows[e, sl] = zvec
            srows[e, sl] = zvec
        return 0

    lax.fori_loop(0, B, zbody, 0)
    for z in range(ROWS_PER_TILE // B):
        pltpu.sync_copy(rows,
                        acc_sh.at[pl.ds(sid * ROWS_PER_TILE + z * B, B), :])
    plsc.subcore_barrier()

    def chunk(i, _):
        cidx = i * NW + wid

        @pl.when(cidx < NCHUNK)
        def _():
            base = cidx * B
            pltpu.sync_copy(gkey.at[pl.ds(base, B)], kbuf)

            def kb(j, _):
                sl = pl.ds(j * LANES, LANES)
                k16 = kbuf[sl]
                gbuf[sl] = lax.shift_right_logical(k16, 3)
                sbuf[sl] = (k16 & 7) * LANES
                return 0

            lax.fori_loop(0, B // LANES, kb, 0)
            cp = pltpu.async_copy(xr.at[gbuf], rows, sem)
            pltpu.sync_copy(c.at[pl.ds(base, B)], cbuf)
            pltpu.sync_copy(dst.at[pl.ds(base, B)], dbuf)
            cp.wait()

            def body(e, _):
                ef = jnp.full((LANES,), e, jnp.int32)
                col = plsc.load_gather(sbuf, [ef]) + iota
                msg = plsc.load_gather(rows, [ef, col])
                cv = plsc.load_gather(cbuf, [ef])
                plsc.store_scatter(srows, [ef, col], msg * cv)
                return 0

            lax.fori_loop(0, B, body, 0)
            pltpu.sync_copy(srows, acc_sh.at[dbuf], add=True)

            def restore(e, _):
                ef = jnp.full((LANES,), e, jnp.int32)
                col = plsc.load_gather(sbuf, [ef]) + iota
                plsc.store_scatter(srows, [ef, col], zvec)
                return 0

            lax.fori_loop(0, B, restore, 0)

        return 0

    lax.fori_loop(0, NITER, chunk, 0)

    plsc.subcore_barrier()
    roff = sid * ROWS_PER_TILE
    for z in range(ROWS_PER_TILE // B):
        pltpu.sync_copy(acc_sh.at[pl.ds(roff + z * B, B), :],
                        part_out.at[cid, pl.ds(roff + z * B, B), :])


def _sc_slot_agg(xr, gkey, dst, c):
    B = 128
    mesh = plsc.VectorSubcoreMesh(**_MESH)
    f = pl.kernel(
        _slot_body,
        out_type=jax.ShapeDtypeStruct((NC, NPAD, 128), jnp.float32),
        mesh=mesh,
        compiler_params=pltpu.CompilerParams(needs_layout_passes=False),
        scratch_types=[
            pltpu.VMEM((B,), jnp.int32),
            pltpu.VMEM((B,), jnp.int32),
            pltpu.VMEM((B,), jnp.int32),
            pltpu.VMEM((B,), jnp.int32),
            pltpu.VMEM((B,), jnp.float32),
            pltpu.VMEM((B, 128), jnp.float32),
            pltpu.VMEM((B, 128), jnp.float32),
            pltpu.SemaphoreType.DMA,
            pltpu.VMEM_SHARED((NPAD, 128), jnp.float32),
        ],
    )
    return f(xr, gkey, dst, c)


def _mm_body(x_ref, w_ref, b_ref, xr_ref, root_ref):
    r = pl.program_id(1)
    acc = jnp.dot(x_ref[...], w_ref[0], preferred_element_type=jnp.float32)

    @pl.when(r < N_REL)
    def _():
        xr_ref[...] = acc

    @pl.when(r == N_REL)
    def _():
        root_ref[...] = acc + b_ref[0]


def _tc_matmul(x, w_all, b, dout, bn):
    n = x.shape[0]
    nb = n // bn
    return pl.pallas_call(
        _mm_body,
        grid=(nb, N_REL + 1),
        in_specs=[
            pl.BlockSpec((bn, x.shape[1]), lambda i, r: (i, 0)),
            pl.BlockSpec((1, x.shape[1], dout), lambda i, r: (r, 0, 0)),
            pl.BlockSpec((1, dout), lambda i, r: (0, 0)),
        ],
        out_specs=[
            pl.BlockSpec((bn, dout),
                         lambda i, r: (jnp.minimum(r, N_REL - 1) * (n // bn) + i, 0)),
            pl.BlockSpec((bn, dout), lambda i, r: (i, 0)),
        ],
        out_shape=[
            jax.ShapeDtypeStruct((N_REL * n, dout), jnp.float32),
            jax.ShapeDtypeStruct((n, dout), jnp.float32),
        ],
    )(x, w_all, b.reshape(1, dout))


def _comb_body(relu, slots, root_ref, p0_ref, p1_ref, o_ref):
    p = p0_ref[0] + p1_ref[0]
    if slots:
        bn = p.shape[0]
        p = p.reshape(bn, 8, p.shape[1] // 8).sum(axis=1)
    v = root_ref[...] + p
    if relu:
        v = jnp.maximum(v, 0.0)
    o_ref[...] = v


def _tc_combine(root, part, relu, bn, slots=False):
    n, d = root.shape
    pd = part.shape[2]
    return pl.pallas_call(
        functools.partial(_comb_body, relu, slots),
        grid=(n // bn,),
        in_specs=[
            pl.BlockSpec((bn, d), lambda i: (i, 0)),
            pl.BlockSpec((1, bn, pd), lambda i: (0, i, 0)),
            pl.BlockSpec((1, bn, pd), lambda i: (1, i, 0)),
        ],
        out_specs=pl.BlockSpec((bn, d), lambda i: (i, 0)),
        out_shape=jax.ShapeDtypeStruct((n, d), jnp.float32),
    )(root, part, part)


def kernel(x, edge_index, edge_type, W_rel1, W_root1, b1, W_rel2, W_root2, b2):
    src = edge_index[0]
    dst = edge_index[1]

    part_cnt, skey, gkey = _sc_count(edge_type, src, dst)
    c = _sc_scale(part_cnt, skey)

    w_all1 = jnp.concatenate([W_rel1, W_root1[None]], axis=0)
    xr1, root1 = _tc_matmul(x, w_all1, b1, 128, 1000)
    part1 = _sc_agg(xr1, gkey, dst, c, D=128)
    h = _tc_combine(root1, part1, relu=True, bn=1000)

    w_all2 = jnp.concatenate([W_rel2, W_root2[None]], axis=0)
    xr2, root2 = _tc_matmul(h, w_all2, b2, 16, 1000)
    part2 = _sc_slot_agg(xr2.reshape(N_NODES, 128), gkey, dst, c)
    logits = _tc_combine(root2, part2, relu=False, bn=1000, slots=True)
    return logits
